# baseline (speedup 1.0000x reference)
import functools

import jax
import jax.numpy as jnp
from jax import lax
from jax.experimental import pallas as pl
from jax.experimental.pallas import tpu as pltpu
from jax.experimental.pallas import tpu_sc as plsc

_NUM_CORES = 2
_NUM_SUBCORES = 16
_NUM_WORKERS = _NUM_CORES * _NUM_SUBCORES
_NBUF = 2
_PACK_ROWS = 8000


def _pack_tables(amplitude_table, phase_table):
    V, D = amplitude_table.shape

    def body(amp_ref, ph_ref, out_ref):
        out_ref[:, :D] = amp_ref[...]
        out_ref[:, D:] = ph_ref[...]

    rows = _PACK_ROWS
    while V % rows:
        rows //= 2
    return pl.pallas_call(
        body,
        grid=(V // rows,),
        in_specs=[
            pl.BlockSpec((rows, D), lambda i: (i, 0)),
            pl.BlockSpec((rows, D), lambda i: (i, 0)),
        ],
        out_specs=pl.BlockSpec((rows, 2 * D), lambda i: (i, 0)),
        out_shape=jax.ShapeDtypeStruct((V, 2 * D), jnp.float32),
        compiler_params=pltpu.CompilerParams(
            dimension_semantics=("parallel",)),
    )(amplitude_table, phase_table)


@jax.jit
def kernel(indices, amplitude_table, phase_table):
    B, H = indices.shape
    V, D = amplitude_table.shape
    N = B * H

    assert N % _NUM_WORKERS == 0
    rows_per_worker = N // _NUM_WORKERS

    chunk = 400
    while rows_per_worker % chunk:
        chunk //= 2
    steps = rows_per_worker // chunk

    idx32 = indices.reshape(N).astype(jnp.int32)
    packed_table = _pack_tables(amplitude_table, phase_table)

    mesh = plsc.VectorSubcoreMesh(core_axis_name="c", subcore_axis_name="s")
    out_sds = jax.ShapeDtypeStruct((N, 2 * D), jnp.float32)

    scratch = (
        [pltpu.VMEM((chunk,), jnp.int32) for _ in range(_NBUF)]
        + [pltpu.VMEM((chunk, 2 * D), jnp.float32) for _ in range(_NBUF)]
        + [pltpu.SemaphoreType.DMA for _ in range(2 * _NBUF)]
    )

    @functools.partial(
        pl.kernel,
        mesh=mesh,
        out_type=out_sds,
        scratch_types=scratch,
        compiler_params=pltpu.CompilerParams(use_tc_tiling_on_sc=False),
    )
    def sc_kernel(idx_hbm, tbl_hbm, out_hbm, *s):
        idx_v = s[0:_NBUF]
        row_v = s[_NBUF:2 * _NBUF]
        sg = s[2 * _NBUF:3 * _NBUF]
        sw = s[3 * _NBUF:4 * _NBUF]

        wid = lax.axis_index("s") * _NUM_CORES + lax.axis_index("c")
        base = wid * rows_per_worker

        def load_and_gather(i, b):
            off = base + i * chunk
            pltpu.sync_copy(idx_hbm.at[pl.ds(off, chunk)], idx_v[b])
            return pltpu.async_copy(tbl_hbm.at[idx_v[b]], row_v[b], sg[b])

        gathers = [None] * _NBUF
        writes = [None] * _NBUF
        gathers[0] = load_and_gather(0, 0)
        for i in range(steps):
            b = i % _NBUF
            if i + 1 < steps:
                nb = (i + 1) % _NBUF
                if writes[nb] is not None:
                    writes[nb].wait()
                    writes[nb] = None
                gathers[nb] = load_and_gather(i + 1, nb)
            gathers[b].wait()
            off = base + i * chunk
            writes[b] = pltpu.async_copy(
                row_v[b], out_hbm.at[pl.ds(off, chunk)], sw[b])
        for w in writes:
            if w is not None:
                w.wait()

    packed = sc_kernel(idx32, packed_table)
    amp_e = packed[:, :D].reshape(B, H, D)
    ph_e = packed[:, D:].reshape(B, H, D)
    return (amp_e, ph_e)


# speedup vs baseline: 1.2784x; 1.2784x over previous
import functools

import jax
import jax.numpy as jnp
from jax import lax
from jax.experimental import pallas as pl
from jax.experimental.pallas import tpu as pltpu
from jax.experimental.pallas import tpu_sc as plsc

_NUM_CORES = 2
_NUM_SUBCORES = 16
_NUM_WORKERS = _NUM_CORES * _NUM_SUBCORES
_NBUF = 2
_HPAD = 56


@jax.jit
def kernel(indices, amplitude_table, phase_table):
    B, H = indices.shape
    V, D = amplitude_table.shape
    N = B * H

    assert N % _NUM_WORKERS == 0
    batches_per_worker = B // _NUM_WORKERS
    bchunk = 8
    while batches_per_worker % bchunk:
        bchunk //= 2
    steps = batches_per_worker // bchunk
    chunk = bchunk * H

    idx32 = indices.reshape(N).astype(jnp.int32)
    packed_table = jnp.concatenate([amplitude_table, phase_table], axis=1)

    mesh = plsc.VectorSubcoreMesh(core_axis_name="c", subcore_axis_name="s")
    out_sds = jax.ShapeDtypeStruct((B * _HPAD, 2 * D), jnp.float32)

    scratch = (
        [pltpu.VMEM((chunk,), jnp.int32) for _ in range(_NBUF)]
        + [pltpu.VMEM((chunk, 2 * D), jnp.float32) for _ in range(_NBUF)]
        + [pltpu.SemaphoreType.DMA for _ in range(2 * _NBUF)]
    )

    @functools.partial(
        pl.kernel,
        mesh=mesh,
        out_type=out_sds,
        scratch_types=scratch,
        compiler_params=pltpu.CompilerParams(use_tc_tiling_on_sc=False),
    )
    def sc_kernel(idx_hbm, tbl_hbm, out_hbm, *s):
        idx_v = s[0:_NBUF]
        row_v = s[_NBUF:2 * _NBUF]
        sg = s[2 * _NBUF:3 * _NBUF]
        sw = s[3 * _NBUF:4 * _NBUF]

        wid = lax.axis_index("s") * _NUM_CORES + lax.axis_index("c")
        batch_base = wid * batches_per_worker
        row_base = batch_base * H

        def load_and_gather(i, b):
            off = row_base + i * chunk
            pltpu.sync_copy(idx_hbm.at[pl.ds(off, chunk)], idx_v[b])
            return pltpu.async_copy(tbl_hbm.at[idx_v[b]], row_v[b], sg[b])

        gathers = [None] * _NBUF
        writes = [None] * _NBUF
        gathers[0] = load_and_gather(0, 0)
        for i in range(steps):
            b = i % _NBUF
            if i + 1 < steps:
                nb = (i + 1) % _NBUF
                if writes[nb] is not None:
                    for w in writes[nb]:
                        w.wait()
                    writes[nb] = None
                gathers[nb] = load_and_gather(i + 1, nb)
            gathers[b].wait()
            ws = []
            for j in range(bchunk):
                dst0 = (batch_base + i * bchunk + j) * _HPAD
                ws.append(pltpu.async_copy(
                    row_v[b].at[pl.ds(j * H, H), :],
                    out_hbm.at[pl.ds(dst0, H)], sw[b]))
            writes[b] = ws
        for ws_ in writes:
            if ws_ is not None:
                for w in ws_:
                    w.wait()

    packed = sc_kernel(idx32, packed_table)
    packed = packed.reshape(B, _HPAD, 2 * D)
    amp_e = packed[:, :H, :D]
    ph_e = packed[:, :H, D:]
    return (amp_e, ph_e)


# speedup vs baseline: 1.5278x; 1.1951x over previous
import functools

import jax
import jax.numpy as jnp
from jax import lax
from jax.experimental import pallas as pl
from jax.experimental.pallas import tpu as pltpu
from jax.experimental.pallas import tpu_sc as plsc

_NUM_CORES = 2
_NUM_SUBCORES = 16
_NUM_WORKERS = _NUM_CORES * _NUM_SUBCORES
_NBUF = 2
_HPAD = 56


@jax.jit
def kernel(indices, amplitude_table, phase_table):
    B, H = indices.shape
    V, D = amplitude_table.shape
    N = B * H

    assert N % _NUM_WORKERS == 0
    batches_per_worker = B // _NUM_WORKERS
    bchunk = 8
    while batches_per_worker % bchunk:
        bchunk //= 2
    steps = batches_per_worker // bchunk
    chunk = bchunk * H

    idx32 = indices.reshape(N).astype(jnp.int32)
    packed_table = jnp.stack(
        [amplitude_table, phase_table], axis=1).reshape(V, 2 * D)

    mesh = plsc.VectorSubcoreMesh(core_axis_name="c", subcore_axis_name="s")
    out_sds = jax.ShapeDtypeStruct((B * _HPAD, 2 * D), jnp.float32)

    scratch = (
        [pltpu.VMEM((chunk,), jnp.int32) for _ in range(_NBUF)]
        + [pltpu.VMEM((chunk, 2 * D), jnp.float32) for _ in range(_NBUF)]
        + [pltpu.SemaphoreType.DMA for _ in range(2 * _NBUF)]
    )

    @functools.partial(
        pl.kernel,
        mesh=mesh,
        out_type=out_sds,
        scratch_types=scratch,
        compiler_params=pltpu.CompilerParams(use_tc_tiling_on_sc=False),
    )
    def sc_kernel(idx_hbm, tbl_hbm, out_hbm, *s):
        idx_v = s[0:_NBUF]
        row_v = s[_NBUF:2 * _NBUF]
        sg = s[2 * _NBUF:3 * _NBUF]
        sw = s[3 * _NBUF:4 * _NBUF]

        wid = lax.axis_index("s") * _NUM_CORES + lax.axis_index("c")
        batch_base = wid * batches_per_worker
        row_base = batch_base * H

        def load_and_gather(i, b):
            off = row_base + i * chunk
            pltpu.sync_copy(idx_hbm.at[pl.ds(off, chunk)], idx_v[b])
            return pltpu.async_copy(tbl_hbm.at[idx_v[b]], row_v[b], sg[b])

        gathers = [None] * _NBUF
        writes = [None] * _NBUF
        gathers[0] = load_and_gather(0, 0)
        for i in range(steps):
            b = i % _NBUF
            if i + 1 < steps:
                nb = (i + 1) % _NBUF
                if writes[nb] is not None:
                    for w in writes[nb]:
                        w.wait()
                    writes[nb] = None
                gathers[nb] = load_and_gather(i + 1, nb)
            gathers[b].wait()
            ws = []
            for j in range(bchunk):
                dst0 = (batch_base + i * bchunk + j) * _HPAD
                ws.append(pltpu.async_copy(
                    row_v[b].at[pl.ds(j * H, H), :],
                    out_hbm.at[pl.ds(dst0, H)], sw[b]))
            writes[b] = ws
        for ws_ in writes:
            if ws_ is not None:
                for w in ws_:
                    w.wait()

    packed = sc_kernel(idx32, packed_table)
    packed = packed.reshape(B, _HPAD, 2 * D)
    amp_e = packed[:, :H, :D]
    ph_e = packed[:, :H, D:]
    return (amp_e, ph_e)


# speedup vs baseline: 1.5301x; 1.0016x over previous
import functools

import jax
import jax.numpy as jnp
from jax import lax
from jax.experimental import pallas as pl
from jax.experimental.pallas import tpu as pltpu
from jax.experimental.pallas import tpu_sc as plsc

_NUM_CORES = 2
_NUM_SUBCORES = 16
_NUM_WORKERS = _NUM_CORES * _NUM_SUBCORES
_NBUF = 3
_HPAD = 56


@jax.jit
def kernel(indices, amplitude_table, phase_table):
    B, H = indices.shape
    V, D = amplitude_table.shape
    N = B * H

    assert N % _NUM_WORKERS == 0
    batches_per_worker = B // _NUM_WORKERS
    bchunk = 4
    while batches_per_worker % bchunk:
        bchunk //= 2
    steps = batches_per_worker // bchunk
    chunk = bchunk * H

    idx32 = indices.reshape(N).astype(jnp.int32)
    packed_table = jnp.stack(
        [amplitude_table, phase_table], axis=1).reshape(V, 2 * D)

    mesh = plsc.VectorSubcoreMesh(core_axis_name="c", subcore_axis_name="s")
    out_sds = jax.ShapeDtypeStruct((B * _HPAD, 2 * D), jnp.float32)

    scratch = (
        [pltpu.VMEM((chunk,), jnp.int32) for _ in range(_NBUF)]
        + [pltpu.VMEM((chunk, 2 * D), jnp.float32) for _ in range(_NBUF)]
        + [pltpu.SemaphoreType.DMA for _ in range(2 * _NBUF)]
    )

    @functools.partial(
        pl.kernel,
        mesh=mesh,
        out_type=out_sds,
        scratch_types=scratch,
        compiler_params=pltpu.CompilerParams(use_tc_tiling_on_sc=False),
    )
    def sc_kernel(idx_hbm, tbl_hbm, out_hbm, *s):
        idx_v = s[0:_NBUF]
        row_v = s[_NBUF:2 * _NBUF]
        sg = s[2 * _NBUF:3 * _NBUF]
        sw = s[3 * _NBUF:4 * _NBUF]

        wid = lax.axis_index("s") * _NUM_CORES + lax.axis_index("c")
        batch_base = wid * batches_per_worker
        row_base = batch_base * H

        def load_and_gather(i, b):
            off = row_base + i * chunk
            pltpu.sync_copy(idx_hbm.at[pl.ds(off, chunk)], idx_v[b])
            return pltpu.async_copy(tbl_hbm.at[idx_v[b]], row_v[b], sg[b])

        gathers = [None] * _NBUF
        writes = [None] * _NBUF
        gathers[0] = load_and_gather(0, 0)
        for i in range(steps):
            b = i % _NBUF
            if i + 1 < steps:
                nb = (i + 1) % _NBUF
                if writes[nb] is not None:
                    for w in writes[nb]:
                        w.wait()
                    writes[nb] = None
                gathers[nb] = load_and_gather(i + 1, nb)
            gathers[b].wait()
            ws = []
            for j in range(bchunk):
                dst0 = (batch_base + i * bchunk + j) * _HPAD
                ws.append(pltpu.async_copy(
                    row_v[b].at[pl.ds(j * H, H), :],
                    out_hbm.at[pl.ds(dst0, H)], sw[b]))
            writes[b] = ws
        for ws_ in writes:
            if ws_ is not None:
                for w in ws_:
                    w.wait()

    packed = sc_kernel(idx32, packed_table)
    packed = packed.reshape(B, _HPAD, 2 * D)
    amp_e = packed[:, :H, :D]
    ph_e = packed[:, :H, D:]
    return (amp_e, ph_e)
